# Initial kernel scaffold; baseline (speedup 1.0000x reference)
import jax
import jax.numpy as jnp
from jax.experimental import pallas as pl


def kernel(x, params):
    raise NotImplementedError("write your pallas kernel here")


# speedup vs baseline: 2.0627x; 2.0627x over previous
import jax
import jax.numpy as jnp
from jax.experimental import pallas as pl
from jax.experimental.pallas import tpu as pltpu

_E = 8
_IN_DIM = 2048
_B = 2048
_HIDDEN = 256
_D_OUT = 64

_NB = 2
_BT = _B // _NB
_CHUNK = 256


def _moe_body(x_ref, W0_ref, b0_ref, W1_ref, b1_ref, Wmu_ref, bmu_ref,
              Wlv_ref, blv_ref, V0_ref, c0_ref, V1_ref, c1_ref,
              Vout_ref, cout_ref,
              mu_out, lv_out, xhat_out, idx_out, best_ref):
    e = pl.program_id(1)
    W0 = W0_ref[0]
    W1 = W1_ref[0]
    Wmu = Wmu_ref[0]
    Wlv = Wlv_ref[0]
    V0 = V0_ref[0]
    V1 = V1_ref[0]
    Vout = Vout_ref[0]
    b0 = b0_ref[0]
    b1 = b1_ref[0]
    bmu = bmu_ref[0]
    blv = blv_ref[0]
    c0 = c0_ref[0]
    c1 = c1_ref[0]
    cout = cout_ref[0]

    for c in range(_BT // _CHUNK):
        sl = pl.ds(c * _CHUNK, _CHUNK)
        x = x_ref[sl, :]
        h = jnp.maximum(jnp.dot(x, W0, preferred_element_type=jnp.float32) + b0, 0.0)
        h = jnp.maximum(jnp.dot(h, W1, preferred_element_type=jnp.float32) + b1, 0.0)
        mu = jnp.dot(h, Wmu, preferred_element_type=jnp.float32) + bmu
        lv = jnp.dot(h, Wlv, preferred_element_type=jnp.float32) + blv
        g = jnp.maximum(jnp.dot(mu, V0, preferred_element_type=jnp.float32) + c0, 0.0)
        g = jnp.maximum(jnp.dot(g, V1, preferred_element_type=jnp.float32) + c1, 0.0)
        xh = jnp.dot(g, Vout, preferred_element_type=jnp.float32) + cout
        d = xh - x
        loss = jnp.mean(d * d, axis=1, keepdims=True)

        @pl.when(e == 0)
        def _():
            best_ref[sl, :] = jnp.full((_CHUNK, 1), jnp.inf, jnp.float32)

        mask = loss < best_ref[sl, :]
        best_ref[sl, :] = jnp.where(mask, loss, best_ref[sl, :])
        mu_out[sl, :] = jnp.where(mask, mu, mu_out[sl, :])
        lv_out[sl, :] = jnp.where(mask, lv, lv_out[sl, :])
        xhat_out[sl, :] = jnp.where(mask, xh, xhat_out[sl, :])
        idx_out[sl, :] = jnp.where(mask, e, idx_out[sl, :])


def kernel(x, params):
    p = params
    grid = (_NB, _E)

    def wspec(shape):
        return pl.BlockSpec((1,) + shape, lambda i, e: (e,) + (0,) * len(shape))

    in_specs = [
        pl.BlockSpec((_BT, _IN_DIM), lambda i, e: (i, 0)),
        wspec((_IN_DIM, _HIDDEN)),
        wspec((1, _HIDDEN)),
        wspec((_HIDDEN, _HIDDEN)),
        wspec((1, _HIDDEN)),
        wspec((_HIDDEN, _D_OUT)),
        wspec((1, _D_OUT)),
        wspec((_HIDDEN, _D_OUT)),
        wspec((1, _D_OUT)),
        wspec((_D_OUT, _HIDDEN)),
        wspec((1, _HIDDEN)),
        wspec((_HIDDEN, _HIDDEN)),
        wspec((1, _HIDDEN)),
        wspec((_HIDDEN, _IN_DIM)),
        wspec((1, _IN_DIM)),
    ]
    out_specs = [
        pl.BlockSpec((_BT, _D_OUT), lambda i, e: (i, 0)),
        pl.BlockSpec((_BT, _D_OUT), lambda i, e: (i, 0)),
        pl.BlockSpec((_BT, _IN_DIM), lambda i, e: (i, 0)),
        pl.BlockSpec((_BT, 1), lambda i, e: (i, 0)),
    ]
    out_shape = [
        jax.ShapeDtypeStruct((_B, _D_OUT), jnp.float32),
        jax.ShapeDtypeStruct((_B, _D_OUT), jnp.float32),
        jax.ShapeDtypeStruct((_B, _IN_DIM), jnp.float32),
        jax.ShapeDtypeStruct((_B, 1), jnp.int32),
    ]

    mu_sel, lv_sel, xhat_sel, idx = pl.pallas_call(
        _moe_body,
        grid=grid,
        in_specs=in_specs,
        out_specs=out_specs,
        out_shape=out_shape,
        scratch_shapes=[pltpu.VMEM((_BT, 1), jnp.float32)],
    )(x,
      p["W0"], p["b0"][:, None, :], p["W1"], p["b1"][:, None, :],
      p["Wmu"], p["bmu"][:, None, :], p["Wlv"], p["blv"][:, None, :],
      p["V0"], p["c0"][:, None, :], p["V1"], p["c1"][:, None, :],
      p["Vout"], p["cout"][:, None, :])

    return (mu_sel, lv_sel, xhat_sel, idx[:, 0])


# speedup vs baseline: 2.0648x; 1.0010x over previous
import jax
import jax.numpy as jnp
from jax.experimental import pallas as pl
from jax.experimental.pallas import tpu as pltpu

_E = 8
_IN_DIM = 2048
_B = 2048
_HIDDEN = 256
_D_OUT = 64

_NB = 2
_BT = _B // _NB
_CHUNK = 256


def _moe_body(x_ref, W0_ref, b0_ref, W1_ref, b1_ref, Wmu_ref, bmu_ref,
              Wlv_ref, blv_ref, V0_ref, c0_ref, V1_ref, c1_ref,
              Vout_ref, cout_ref,
              mu_out, lv_out, xhat_out, idx_out, best_ref):
    e = pl.program_id(1)
    W0 = W0_ref[0]
    W1 = W1_ref[0]
    Wmu = Wmu_ref[0]
    Wlv = Wlv_ref[0]
    V0 = V0_ref[0]
    V1 = V1_ref[0]
    Vout = Vout_ref[0]
    b0 = b0_ref[0]
    b1 = b1_ref[0]
    bmu = bmu_ref[0]
    blv = blv_ref[0]
    c0 = c0_ref[0]
    c1 = c1_ref[0]
    cout = cout_ref[0]

    for c in range(_BT // _CHUNK):
        sl = pl.ds(c * _CHUNK, _CHUNK)
        x = x_ref[sl, :]
        h = jnp.maximum(jnp.dot(x, W0, preferred_element_type=jnp.float32) + b0, 0.0)
        h = jnp.maximum(jnp.dot(h, W1, preferred_element_type=jnp.float32) + b1, 0.0)
        mu = jnp.dot(h, Wmu, preferred_element_type=jnp.float32) + bmu
        lv = jnp.dot(h, Wlv, preferred_element_type=jnp.float32) + blv
        g = jnp.maximum(jnp.dot(mu, V0, preferred_element_type=jnp.float32) + c0, 0.0)
        g = jnp.maximum(jnp.dot(g, V1, preferred_element_type=jnp.float32) + c1, 0.0)
        xh = jnp.dot(g, Vout, preferred_element_type=jnp.float32) + cout
        d = xh - x
        loss = jnp.mean(d * d, axis=1, keepdims=True)

        @pl.when(e == 0)
        def _():
            best_ref[sl, :] = jnp.full((_CHUNK, 1), jnp.inf, jnp.float32)

        mask = loss < best_ref[sl, :]
        best_ref[sl, :] = jnp.where(mask, loss, best_ref[sl, :])
        mu_out[sl, :] = jnp.where(mask, mu, mu_out[sl, :])
        lv_out[sl, :] = jnp.where(mask, lv, lv_out[sl, :])
        xhat_out[sl, :] = jnp.where(mask, xh, xhat_out[sl, :])
        idx_out[sl, :] = jnp.where(mask, e, idx_out[sl, :])


def kernel(x, params):
    p = params
    grid = (_NB, _E)

    def wspec(shape):
        return pl.BlockSpec((1,) + shape, lambda i, e: (e,) + (0,) * len(shape))

    in_specs = [
        pl.BlockSpec((_BT, _IN_DIM), lambda i, e: (i, 0)),
        wspec((_IN_DIM, _HIDDEN)),
        wspec((1, _HIDDEN)),
        wspec((_HIDDEN, _HIDDEN)),
        wspec((1, _HIDDEN)),
        wspec((_HIDDEN, _D_OUT)),
        wspec((1, _D_OUT)),
        wspec((_HIDDEN, _D_OUT)),
        wspec((1, _D_OUT)),
        wspec((_D_OUT, _HIDDEN)),
        wspec((1, _HIDDEN)),
        wspec((_HIDDEN, _HIDDEN)),
        wspec((1, _HIDDEN)),
        wspec((_HIDDEN, _IN_DIM)),
        wspec((1, _IN_DIM)),
    ]
    out_specs = [
        pl.BlockSpec((_BT, _D_OUT), lambda i, e: (i, 0)),
        pl.BlockSpec((_BT, _D_OUT), lambda i, e: (i, 0)),
        pl.BlockSpec((_BT, _IN_DIM), lambda i, e: (i, 0)),
        pl.BlockSpec((_BT, 1), lambda i, e: (i, 0)),
    ]
    out_shape = [
        jax.ShapeDtypeStruct((_B, _D_OUT), jnp.float32),
        jax.ShapeDtypeStruct((_B, _D_OUT), jnp.float32),
        jax.ShapeDtypeStruct((_B, _IN_DIM), jnp.float32),
        jax.ShapeDtypeStruct((_B, 1), jnp.int32),
    ]

    mu_sel, lv_sel, xhat_sel, idx = pl.pallas_call(
        _moe_body,
        grid=grid,
        in_specs=in_specs,
        out_specs=out_specs,
        out_shape=out_shape,
        scratch_shapes=[pltpu.VMEM((_BT, 1), jnp.float32)],
        compiler_params=pltpu.CompilerParams(
            dimension_semantics=("parallel", "arbitrary")),
    )(x,
      p["W0"], p["b0"][:, None, :], p["W1"], p["b1"][:, None, :],
      p["Wmu"], p["bmu"][:, None, :], p["Wlv"], p["blv"][:, None, :],
      p["V0"], p["c0"][:, None, :], p["V1"], p["c1"][:, None, :],
      p["Vout"], p["cout"][:, None, :])

    return (mu_sel, lv_sel, xhat_sel, idx[:, 0])
